# Initial kernel scaffold; baseline (speedup 1.0000x reference)
import jax
import jax.numpy as jnp
from jax.experimental import pallas as pl


def kernel(x, edge_index_in, edge_index_out, params):
    raise NotImplementedError("write your pallas kernel here")


# speedup vs baseline: 1.0846x; 1.0846x over previous
import functools
import math

import jax
import jax.numpy as jnp
from jax.experimental import pallas as pl
from jax.experimental.pallas import tpu as pltpu

_N = 10000
_D = 128
_AVG_LOG = math.log(33.0)
_ROW_BLK = 1000


def _pre_body(x_ref, w_ref, out_ref):
    out_ref[...] = jnp.dot(x_ref[...], w_ref[...],
                           preferred_element_type=jnp.float32)


def _pre_matmul(x, w):
    n = x.shape[0]
    k = w.shape[1]
    grid = n // _ROW_BLK
    return pl.pallas_call(
        _pre_body,
        grid=(grid,),
        in_specs=[
            pl.BlockSpec((_ROW_BLK, _D), lambda i: (i, 0)),
            pl.BlockSpec((_D, k), lambda i: (0, 0)),
        ],
        out_specs=pl.BlockSpec((_ROW_BLK, k), lambda i: (i, 0)),
        out_shape=jax.ShapeDtypeStruct((n, k), jnp.float32),
    )(x, w)


def _combine_body(x_ref, deg_i_ref, s1_i_ref, s2_i_ref, mn_i_ref, mx_i_ref,
                  c_i_ref, deg_o_ref, s1_o_ref, s2_o_ref, mn_o_ref, mx_o_ref,
                  c_o_ref, wpost_i_ref, bpost_i_ref, wlin_i_ref, blin_i_ref,
                  wpost_o_ref, bpost_o_ref, wlin_o_ref, blin_o_ref,
                  wcomb_ref, bcomb_ref, wout_ref, bout_ref, y_ref):
    x = x_ref[...]

    def conv_half(deg_ref, s1_ref, s2_ref, mn_ref, mx_ref, c_ref,
                  wpost_ref, bpost_ref, wlin_ref, blin_ref):
        deg = deg_ref[...]
        s1 = s1_ref[...]
        s2 = s2_ref[...]
        c = c_ref[...]
        deg_c = jnp.maximum(deg, 1.0)
        ssum = deg * c + s1
        mean = ssum / deg_c
        msq = (deg * c * c + 2.0 * c * s1 + s2) / deg_c
        std = jnp.sqrt(jnp.maximum(msq - mean * mean, 0.0) + 1e-5)
        has = deg > 0.0
        mn = jnp.where(has, c + mn_ref[...], 0.0)
        mx = jnp.where(has, c + mx_ref[...], 0.0)
        agg = jnp.concatenate([mean, ssum, std, mn, mx], axis=-1)
        ld = jnp.log(deg_c + 1.0)
        sc1 = ld * (1.0 / _AVG_LOG)
        sc2 = _AVG_LOG / ld
        full = jnp.concatenate([x, agg, agg * sc1, agg * sc2], axis=-1)
        h = jnp.dot(full, wpost_ref[...],
                    preferred_element_type=jnp.float32) + bpost_ref[...]
        h = jnp.dot(h, wlin_ref[...],
                    preferred_element_type=jnp.float32) + blin_ref[...]
        return jnp.maximum(h, 0.0)

    h_in = conv_half(deg_i_ref, s1_i_ref, s2_i_ref, mn_i_ref, mx_i_ref,
                     c_i_ref, wpost_i_ref, bpost_i_ref, wlin_i_ref, blin_i_ref)
    h_out = conv_half(deg_o_ref, s1_o_ref, s2_o_ref, mn_o_ref, mx_o_ref,
                      c_o_ref, wpost_o_ref, bpost_o_ref, wlin_o_ref,
                      blin_o_ref)
    hcat = jnp.concatenate([x, h_in, h_out], axis=-1)
    h = jnp.dot(hcat, wcomb_ref[...],
                preferred_element_type=jnp.float32) + bcomb_ref[...]
    h = jnp.maximum(h, 0.0)
    y_ref[...] = jnp.dot(h, wout_ref[...],
                         preferred_element_type=jnp.float32) + bout_ref[...]


def _combine(x, stats_in, stats_out, params):
    n = x.shape[0]
    grid = n // _ROW_BLK
    row = lambda i: (i, 0)
    rep = lambda i: (0, 0)

    def specs_for(st):
        deg, s1, s2, mn, mx, c = st
        return [
            pl.BlockSpec((_ROW_BLK, 1), row),
            pl.BlockSpec((_ROW_BLK, _D), row),
            pl.BlockSpec((_ROW_BLK, _D), row),
            pl.BlockSpec((_ROW_BLK, _D), row),
            pl.BlockSpec((_ROW_BLK, _D), row),
            pl.BlockSpec((_ROW_BLK, _D), row),
        ]

    ps, pd = params['supply'], params['demand']
    weight_args = [ps['post_W'], ps['post_b'][None, :], ps['lin_W'],
                   ps['lin_b'][None, :], pd['post_W'], pd['post_b'][None, :],
                   pd['lin_W'], pd['lin_b'][None, :], params['comb_W'],
                   params['comb_b'][None, :], params['out_W'],
                   params['out_b'][None, :]]
    weight_specs = [pl.BlockSpec(w.shape, rep) for w in weight_args]

    return pl.pallas_call(
        _combine_body,
        grid=(grid,),
        in_specs=([pl.BlockSpec((_ROW_BLK, _D), row)] + specs_for(stats_in)
                  + specs_for(stats_out) + weight_specs),
        out_specs=pl.BlockSpec((_ROW_BLK, 64), row),
        out_shape=jax.ShapeDtypeStruct((n, 64), jnp.float32),
    )(x, *stats_in, *stats_out, *weight_args)


def _segment_stats(b, edge_index):
    src = edge_index[0]
    dst = edge_index[1]
    bs = b[src]
    deg = jnp.zeros((_N,), jnp.float32).at[dst].add(1.0)
    s1 = jax.ops.segment_sum(bs, dst, num_segments=_N)
    s2 = jax.ops.segment_sum(bs * bs, dst, num_segments=_N)
    mn = jax.ops.segment_min(bs, dst, num_segments=_N)
    mx = jax.ops.segment_max(bs, dst, num_segments=_N)
    mn = jnp.where(jnp.isfinite(mn), mn, 0.0)
    mx = jnp.where(jnp.isfinite(mx), mx, 0.0)
    return deg[:, None], s1, s2, mn, mx


def kernel(x, edge_index_in, edge_index_out, params):
    ps, pd = params['supply'], params['demand']
    w_all = jnp.concatenate([
        ps['pre_W'][:_D], ps['pre_W'][_D:], pd['pre_W'][:_D], pd['pre_W'][_D:],
    ], axis=1)
    pre = _pre_matmul(x, w_all)
    c_in = pre[:, :_D] + ps['pre_b'][None, :]
    b_in = pre[:, _D:2 * _D]
    c_out = pre[:, 2 * _D:3 * _D] + pd['pre_b'][None, :]
    b_out = pre[:, 3 * _D:]

    deg_i, s1_i, s2_i, mn_i, mx_i = _segment_stats(b_in, edge_index_in)
    deg_o, s1_o, s2_o, mn_o, mx_o = _segment_stats(b_out, edge_index_out)

    return _combine(x,
                    (deg_i, s1_i, s2_i, mn_i, mx_i, c_in),
                    (deg_o, s1_o, s2_o, mn_o, mx_o, c_out),
                    params)
